# Initial kernel scaffold; baseline (speedup 1.0000x reference)
import jax
import jax.numpy as jnp
from jax.experimental import pallas as pl


def kernel(token_ids, lookup):
    raise NotImplementedError("write your pallas kernel here")


# speedup vs baseline: 2.9768x; 2.9768x over previous
import functools

import jax
import jax.numpy as jnp
from jax import lax
from jax.experimental import pallas as pl
from jax.experimental.pallas import tpu as pltpu
from jax.experimental.pallas import tpu_sc as plsc

NUM_EMB = 100000
D = 128
BATCH = 4096
HIST = 50
TOTAL = BATCH * HIST

NC = 2
NS = 16
NW = NC * NS
PER_W = TOTAL // NW
CHUNK = 128
NCH = PER_W // CHUNK


@functools.partial(
    pl.kernel,
    mesh=plsc.VectorSubcoreMesh(core_axis_name="c", subcore_axis_name="s"),
    out_type=jax.ShapeDtypeStruct((NW, PER_W, D), jnp.float32),
    scratch_types=[
        pltpu.VMEM((NCH, CHUNK), jnp.int32),
        pltpu.VMEM((CHUNK, D), jnp.float32),
        pltpu.SemaphoreType.DMA,
    ],
)
def _emb_gather(idx_hbm, table_hbm, out_hbm, idx_v, buf, sem):
    wid = lax.axis_index("s") * NC + lax.axis_index("c")
    pltpu.sync_copy(idx_hbm.at[wid], idx_v)

    def step(g, carry):
        pltpu.async_copy(table_hbm.at[idx_v.at[g]], buf, sem).wait()
        pltpu.sync_copy(buf, out_hbm.at[wid, pl.ds(g * CHUNK, CHUNK)])
        return carry

    lax.fori_loop(0, NCH, step, 0)


def kernel(token_ids, lookup):
    idx = token_ids.reshape(NW, NCH, CHUNK).astype(jnp.int32)
    out = _emb_gather(idx, lookup)
    return out.reshape(BATCH, HIST, D)


# speedup vs baseline: 3.3546x; 1.1269x over previous
import functools

import jax
import jax.numpy as jnp
from jax import lax
from jax.experimental import pallas as pl
from jax.experimental.pallas import tpu as pltpu
from jax.experimental.pallas import tpu_sc as plsc

NUM_EMB = 100000
D = 128
BATCH = 4096
HIST = 50
TOTAL = BATCH * HIST

NC = 2
NS = 16
NW = NC * NS
PER_W = TOTAL // NW
CHUNK = 128
NCH = PER_W // CHUNK
NBUF = 4


@functools.partial(
    pl.kernel,
    mesh=plsc.VectorSubcoreMesh(core_axis_name="c", subcore_axis_name="s"),
    out_type=jax.ShapeDtypeStruct((NW, PER_W, D), jnp.float32),
    scratch_types=[
        pltpu.VMEM((NCH, CHUNK), jnp.int32),
        pltpu.VMEM((NBUF, CHUNK, D), jnp.float32),
        pltpu.SemaphoreType.DMA,
        pltpu.SemaphoreType.DMA,
    ],
)
def _emb_gather(idx_hbm, table_hbm, out_hbm, idx_v, buf, gsem, ssem):
    wid = lax.axis_index("s") * NC + lax.axis_index("c")
    pltpu.sync_copy(idx_hbm.at[wid], idx_v)

    for b in range(NBUF - 1):
        pltpu.async_copy(table_hbm.at[idx_v.at[b]], buf.at[b], gsem)

    def step(g, carry):
        slot = lax.rem(g, NBUF)
        pltpu.make_async_copy(
            table_hbm.at[idx_v.at[g]], buf.at[slot], gsem
        ).wait()

        @pl.when(g + NBUF - 1 < NCH)
        def _():
            nslot = lax.rem(g + NBUF - 1, NBUF)

            @pl.when(g >= 1)
            def _():
                pltpu.make_async_copy(
                    buf.at[nslot],
                    out_hbm.at[wid, pl.ds((g - 1) * CHUNK, CHUNK)],
                    ssem,
                ).wait()

            pltpu.async_copy(
                table_hbm.at[idx_v.at[g + NBUF - 1]], buf.at[nslot], gsem
            )

        pltpu.async_copy(
            buf.at[slot], out_hbm.at[wid, pl.ds(g * CHUNK, CHUNK)], ssem
        )
        return carry

    lax.fori_loop(0, NCH, step, 0)

    for g in range(NCH - NBUF, NCH):
        pltpu.make_async_copy(
            buf.at[g % NBUF], out_hbm.at[wid, pl.ds(g * CHUNK, CHUNK)], ssem
        ).wait()


def kernel(token_ids, lookup):
    idx = token_ids.reshape(NW, NCH, CHUNK).astype(jnp.int32)
    out = _emb_gather(idx, lookup)
    return out.reshape(BATCH, HIST, D)


# speedup vs baseline: 5.9143x; 1.7630x over previous
import functools

import jax
import jax.numpy as jnp
from jax import lax
from jax.experimental import pallas as pl
from jax.experimental.pallas import tpu as pltpu
from jax.experimental.pallas import tpu_sc as plsc

NUM_EMB = 100000
D = 128
BATCH = 4096
HIST = 50

NC = 2
NS = 16
NW = NC * NS
ROWS_W = BATCH // NW
NBUF = 4


@functools.partial(
    pl.kernel,
    mesh=plsc.VectorSubcoreMesh(core_axis_name="c", subcore_axis_name="s"),
    out_type=jax.ShapeDtypeStruct((BATCH, HIST, D), jnp.float32),
    scratch_types=[
        pltpu.VMEM((ROWS_W, HIST), jnp.int32),
        pltpu.VMEM((NBUF, HIST, D), jnp.float32),
        pltpu.SemaphoreType.DMA,
        pltpu.SemaphoreType.DMA,
    ],
)
def _emb_gather(idx_hbm, table_hbm, out_hbm, idx_v, buf, gsem, ssem):
    wid = lax.axis_index("s") * NC + lax.axis_index("c")
    base = wid * ROWS_W
    pltpu.sync_copy(idx_hbm.at[pl.ds(base, ROWS_W)], idx_v)

    for b in range(NBUF - 1):
        pltpu.async_copy(table_hbm.at[idx_v.at[b]], buf.at[b], gsem)

    def step(g, carry):
        slot = lax.rem(g, NBUF)
        pltpu.make_async_copy(
            table_hbm.at[idx_v.at[g]], buf.at[slot], gsem
        ).wait()

        @pl.when(g + NBUF - 1 < ROWS_W)
        def _():
            nslot = lax.rem(g + NBUF - 1, NBUF)

            @pl.when(g >= 1)
            def _():
                pltpu.make_async_copy(
                    buf.at[nslot], out_hbm.at[base + g - 1], ssem
                ).wait()

            pltpu.async_copy(
                table_hbm.at[idx_v.at[g + NBUF - 1]], buf.at[nslot], gsem
            )

        pltpu.async_copy(buf.at[slot], out_hbm.at[base + g], ssem)
        return carry

    lax.fori_loop(0, ROWS_W, step, 0)

    for g in range(ROWS_W - NBUF, ROWS_W):
        pltpu.make_async_copy(
            buf.at[g % NBUF], out_hbm.at[base + g], ssem
        ).wait()


def kernel(token_ids, lookup):
    return _emb_gather(token_ids.astype(jnp.int32), lookup)


# speedup vs baseline: 5.9171x; 1.0005x over previous
import functools

import jax
import jax.numpy as jnp
from jax import lax
from jax.experimental import pallas as pl
from jax.experimental.pallas import tpu as pltpu
from jax.experimental.pallas import tpu_sc as plsc

NUM_EMB = 100000
D = 128
BATCH = 4096
HIST = 50

NC = 2
NS = 16
NW = NC * NS
ROWS_W = BATCH // NW
NBUF = 4


@functools.partial(
    pl.kernel,
    mesh=plsc.VectorSubcoreMesh(core_axis_name="c", subcore_axis_name="s"),
    compiler_params=pltpu.CompilerParams(use_tc_tiling_on_sc=True),
    out_type=jax.ShapeDtypeStruct((BATCH, HIST, D), jnp.float32),
    scratch_types=[
        pltpu.VMEM((ROWS_W, HIST), jnp.int32),
        pltpu.VMEM((NBUF, HIST, D), jnp.float32),
        pltpu.SemaphoreType.DMA,
        pltpu.SemaphoreType.DMA,
    ],
)
def _emb_gather(idx_hbm, table_hbm, out_hbm, idx_v, buf, gsem, ssem):
    wid = lax.axis_index("s") * NC + lax.axis_index("c")
    base = wid * ROWS_W
    pltpu.sync_copy(idx_hbm.at[pl.ds(base, ROWS_W)], idx_v)

    for b in range(NBUF - 1):
        pltpu.async_copy(table_hbm.at[idx_v.at[b]], buf.at[b], gsem)

    def step(g, carry):
        slot = lax.rem(g, NBUF)
        pltpu.make_async_copy(
            table_hbm.at[idx_v.at[g]], buf.at[slot], gsem
        ).wait()

        @pl.when(g + NBUF - 1 < ROWS_W)
        def _():
            nslot = lax.rem(g + NBUF - 1, NBUF)

            @pl.when(g >= 1)
            def _():
                pltpu.make_async_copy(
                    buf.at[nslot], out_hbm.at[base + g - 1], ssem
                ).wait()

            pltpu.async_copy(
                table_hbm.at[idx_v.at[g + NBUF - 1]], buf.at[nslot], gsem
            )

        pltpu.async_copy(buf.at[slot], out_hbm.at[base + g], ssem)
        return carry

    lax.fori_loop(0, ROWS_W, step, 0)

    for g in range(ROWS_W - NBUF, ROWS_W):
        pltpu.make_async_copy(
            buf.at[g % NBUF], out_hbm.at[base + g], ssem
        ).wait()


def kernel(token_ids, lookup):
    return _emb_gather(token_ids.astype(jnp.int32), lookup)


# speedup vs baseline: 10.7847x; 1.8226x over previous
import functools

import jax
import jax.numpy as jnp
from jax import lax
from jax.experimental import pallas as pl
from jax.experimental.pallas import tpu as pltpu
from jax.experimental.pallas import tpu_sc as plsc

NUM_EMB = 100000
D = 128
BATCH = 4096
HIST = 50

NC = 2
NS = 16
NW = NC * NS
COLS_W = BATCH // NW
NBUF = 4


@functools.partial(
    pl.kernel,
    mesh=plsc.VectorSubcoreMesh(core_axis_name="c", subcore_axis_name="s"),
    out_type=jax.ShapeDtypeStruct((HIST, BATCH, D), jnp.float32),
    scratch_types=[
        pltpu.VMEM((HIST, COLS_W), jnp.int32),
        pltpu.VMEM((NBUF, COLS_W, D), jnp.float32),
        pltpu.SemaphoreType.DMA,
        pltpu.SemaphoreType.DMA,
    ],
)
def _emb_gather(idx_hbm, table_hbm, out_hbm, idx_v, buf, gsem, ssem):
    wid = lax.axis_index("s") * NC + lax.axis_index("c")
    base = wid * COLS_W
    pltpu.sync_copy(idx_hbm.at[:, pl.ds(base, COLS_W)], idx_v)

    for b in range(NBUF - 1):
        pltpu.async_copy(table_hbm.at[idx_v.at[b]], buf.at[b], gsem)

    def step(g, carry):
        slot = lax.rem(g, NBUF)
        pltpu.make_async_copy(
            table_hbm.at[idx_v.at[g]], buf.at[slot], gsem
        ).wait()

        @pl.when(g + NBUF - 1 < HIST)
        def _():
            nslot = lax.rem(g + NBUF - 1, NBUF)

            @pl.when(g >= 1)
            def _():
                pltpu.make_async_copy(
                    buf.at[nslot],
                    out_hbm.at[g - 1, pl.ds(base, COLS_W)],
                    ssem,
                ).wait()

            pltpu.async_copy(
                table_hbm.at[idx_v.at[g + NBUF - 1]], buf.at[nslot], gsem
            )

        pltpu.async_copy(
            buf.at[slot], out_hbm.at[g, pl.ds(base, COLS_W)], ssem
        )
        return carry

    lax.fori_loop(0, HIST, step, 0)

    for g in range(HIST - NBUF, HIST):
        pltpu.make_async_copy(
            buf.at[g % NBUF], out_hbm.at[g, pl.ds(base, COLS_W)], ssem
        ).wait()


def kernel(token_ids, lookup):
    out = _emb_gather(token_ids.T.astype(jnp.int32), lookup)
    return out.transpose(1, 0, 2)
